# baseline (speedup 1.0000x reference)
import functools

import jax
import jax.numpy as jnp
import numpy as np
from jax.experimental import pallas as pl
from jax.experimental.pallas import tpu as pltpu

_EPS = 1e-5
_VMEM_LIMIT = 48 * 1024 * 1024


def _proj_mix_kernel(x_ref, g_ref, w_ref, w1_ref, s_ref, m_ref, z_ref,
                     st_ref):
    x = x_ref[0]
    a = jnp.dot(w_ref[...], x, preferred_element_type=jnp.float32)
    c = jnp.dot(w1_ref[...], x, preferred_element_type=jnp.float32)
    gb = jnp.dot(s_ref[...], g_ref[0],
                 preferred_element_type=jnp.float32) * m_ref[...]
    z = jnp.dot(a, gb, preferred_element_type=jnp.float32) + c
    z_ref[0] = z.astype(z_ref.dtype)
    s1 = jnp.sum(z, axis=1, keepdims=True)
    s2 = jnp.sum(z * z, axis=1, keepdims=True)
    st_ref[0] = jnp.concatenate([s1, s2], axis=1).astype(st_ref.dtype)


def _bn_relu_kernel(n_total, z_ref, st_ref, ga_ref, be_ref, y_ref):
    tot = jnp.sum(st_ref[...].astype(jnp.float32), axis=0)
    inv_n = 1.0 / n_total
    mean = tot[:, 0:1] * inv_n
    var = jnp.maximum(tot[:, 1:2] * inv_n - mean * mean, 0.0)
    inv = jax.lax.rsqrt(var + _EPS)
    scale = inv * ga_ref[...].astype(jnp.float32)
    shift = be_ref[...].astype(jnp.float32) - mean * scale
    z = z_ref[0].astype(jnp.float32)
    y_ref[0] = jnp.maximum(z * scale + shift, 0.0).astype(y_ref.dtype)


@jax.jit
def _forward(x1, g, w, w1, gamma, beta):
    B, Cin, J, T = x1.shape
    Cout = w.shape[0]
    JT = J * T

    x2 = x1.reshape(B, Cin, JT)
    gflat = jnp.transpose(g, (0, 3, 2, 1)).reshape(B, J, JT)

    r = np.arange(JT)
    s_sel = jnp.asarray(
        (r[:, None] // T == np.arange(J)[None, :]).astype(np.float32))
    eqmask = jnp.asarray(
        (r[:, None] % T == r[None, :] % T).astype(np.float32))

    z, stats = pl.pallas_call(
        _proj_mix_kernel,
        out_shape=(
            jax.ShapeDtypeStruct((B, Cout, JT), jnp.float32),
            jax.ShapeDtypeStruct((B, Cout, 2), jnp.float32),
        ),
        grid=(B,),
        in_specs=[
            pl.BlockSpec((1, Cin, JT), lambda b: (b, 0, 0)),
            pl.BlockSpec((1, J, JT), lambda b: (b, 0, 0)),
            pl.BlockSpec((Cout, Cin), lambda b: (0, 0)),
            pl.BlockSpec((Cout, Cin), lambda b: (0, 0)),
            pl.BlockSpec((JT, J), lambda b: (0, 0)),
            pl.BlockSpec((JT, JT), lambda b: (0, 0)),
        ],
        out_specs=(
            pl.BlockSpec((1, Cout, JT), lambda b: (b, 0, 0)),
            pl.BlockSpec((1, Cout, 2), lambda b: (b, 0, 0)),
        ),
        compiler_params=pltpu.CompilerParams(
            dimension_semantics=("parallel",),
            vmem_limit_bytes=_VMEM_LIMIT,
        ),
    )(x2, gflat, w, w1, s_sel, eqmask)

    y = pl.pallas_call(
        functools.partial(_bn_relu_kernel, float(B * JT)),
        out_shape=jax.ShapeDtypeStruct((B, Cout, JT), x1.dtype),
        grid=(B,),
        in_specs=[
            pl.BlockSpec((1, Cout, JT), lambda b: (b, 0, 0)),
            pl.BlockSpec((B, Cout, 2), lambda b: (0, 0, 0)),
            pl.BlockSpec((Cout, 1), lambda b: (0, 0)),
            pl.BlockSpec((Cout, 1), lambda b: (0, 0)),
        ],
        out_specs=pl.BlockSpec((1, Cout, JT), lambda b: (b, 0, 0)),
        compiler_params=pltpu.CompilerParams(
            dimension_semantics=("parallel",),
            vmem_limit_bytes=_VMEM_LIMIT,
        ),
    )(z, stats, gamma.reshape(Cout, 1), beta.reshape(Cout, 1))

    return y.reshape(B, Cout, J, T)


def kernel(x1, g, w, w1, b1, gamma, beta):
    del b1
    return _forward(x1, g, w, w1, gamma, beta)


# speedup vs baseline: 1.9142x; 1.9142x over previous
import functools

import jax
import jax.numpy as jnp
import numpy as np
from jax.experimental import pallas as pl
from jax.experimental.pallas import tpu as pltpu

_EPS = 1e-5
_VMEM_LIMIT = 48 * 1024 * 1024


def _proj_mix_kernel(x_ref, g_ref, wcatt_ref, e_ref, bm_ref, z_ref, st_ref):
    jdim, _, bdim, cin = x_ref.shape
    cout = z_ref.shape[3]
    rows = jdim * bdim
    x2d = x_ref[...].reshape(rows, cin)
    p = jnp.dot(x2d, wcatt_ref[...], preferred_element_type=jnp.float32)
    a = p[:, :cout]
    c = p[:, cout:]
    g2d = g_ref[...].reshape(rows, g_ref.shape[3])
    m_t = jnp.dot(g2d, e_ref[...],
                  preferred_element_type=jnp.float32) * bm_ref[...]
    z = jnp.dot(m_t, a, preferred_element_type=jnp.float32) + c
    z_ref[...] = z.reshape(jdim, 1, bdim, cout).astype(z_ref.dtype)
    s1 = jnp.sum(z, axis=0, keepdims=True)
    s2 = jnp.sum(z * z, axis=0, keepdims=True)
    st_ref[...] = jnp.concatenate([s1, s2], axis=0).reshape(1, 2, cout)


def _bn_relu_kernel(n_total, z_ref, st_ref, ga_ref, be_ref, y_ref):
    jdim, tdim, bdim, cout = z_ref.shape
    tot = jnp.sum(st_ref[...], axis=0)
    inv_n = 1.0 / n_total
    mean = tot[0:1] * inv_n
    var = jnp.maximum(tot[1:2] * inv_n - mean * mean, 0.0)
    inv = jax.lax.rsqrt(var + _EPS)
    scale = inv * ga_ref[...]
    shift = be_ref[...] - mean * scale
    z = z_ref[...].reshape(jdim * tdim * bdim, cout)
    y = jnp.maximum(z * scale + shift, 0.0)
    y_ref[...] = y.reshape(jdim, tdim, bdim, cout).astype(y_ref.dtype)


@jax.jit
def _forward(x1, g, w, w1, gamma, beta):
    B, Cin, J, T = x1.shape
    Cout = w.shape[0]

    xp = jnp.transpose(x1, (2, 3, 0, 1))
    gp = jnp.transpose(g, (1, 2, 0, 3))
    wcatt = jnp.concatenate([w.T, w1.T], axis=1)

    col = np.arange(J * B)
    e_sel = jnp.asarray(
        (np.arange(J)[:, None] == col[None, :] // B).astype(np.float32))
    bmask = jnp.asarray(
        (col[:, None] % B == col[None, :] % B).astype(np.float32))

    z, stats = pl.pallas_call(
        _proj_mix_kernel,
        out_shape=(
            jax.ShapeDtypeStruct((J, T, B, Cout), jnp.float32),
            jax.ShapeDtypeStruct((T, 2, Cout), jnp.float32),
        ),
        grid=(T,),
        in_specs=[
            pl.BlockSpec((J, 1, B, Cin), lambda t: (0, t, 0, 0)),
            pl.BlockSpec((1, J, B, J), lambda t: (t, 0, 0, 0)),
            pl.BlockSpec((Cin, 2 * Cout), lambda t: (0, 0)),
            pl.BlockSpec((J, J * B), lambda t: (0, 0)),
            pl.BlockSpec((J * B, J * B), lambda t: (0, 0)),
        ],
        out_specs=(
            pl.BlockSpec((J, 1, B, Cout), lambda t: (0, t, 0, 0)),
            pl.BlockSpec((1, 2, Cout), lambda t: (t, 0, 0)),
        ),
        compiler_params=pltpu.CompilerParams(
            dimension_semantics=("parallel",),
            vmem_limit_bytes=_VMEM_LIMIT,
        ),
    )(xp, gp, wcatt, e_sel, bmask)

    y = pl.pallas_call(
        functools.partial(_bn_relu_kernel, float(B * J * T)),
        out_shape=jax.ShapeDtypeStruct((J, T, B, Cout), x1.dtype),
        grid=(J,),
        in_specs=[
            pl.BlockSpec((1, T, B, Cout), lambda j: (j, 0, 0, 0)),
            pl.BlockSpec((T, 2, Cout), lambda j: (0, 0, 0)),
            pl.BlockSpec((1, Cout), lambda j: (0, 0)),
            pl.BlockSpec((1, Cout), lambda j: (0, 0)),
        ],
        out_specs=pl.BlockSpec((1, T, B, Cout), lambda j: (j, 0, 0, 0)),
        compiler_params=pltpu.CompilerParams(
            dimension_semantics=("parallel",),
            vmem_limit_bytes=_VMEM_LIMIT,
        ),
    )(z, stats, gamma.reshape(1, Cout), beta.reshape(1, Cout))

    return jnp.transpose(y, (2, 3, 0, 1))


def kernel(x1, g, w, w1, b1, gamma, beta):
    del b1
    return _forward(x1, g, w, w1, gamma, beta)


# speedup vs baseline: 2.9690x; 1.5510x over previous
import functools

import jax
import jax.numpy as jnp
import numpy as np
from jax.experimental import pallas as pl
from jax.experimental.pallas import tpu as pltpu

_EPS = 1e-5
_VMEM_LIMIT = 48 * 1024 * 1024


def _proj_mix_kernel(x_ref, g_ref, wcatt_ref, e_ref, bm_ref, z_ref, st_ref):
    jdim, tc, bdim, cin = x_ref.shape
    cout = z_ref.shape[3]
    rows = jdim * bdim
    x2d = x_ref[...].reshape(jdim * tc * bdim, cin)
    p = jnp.dot(x2d, wcatt_ref[...], preferred_element_type=jnp.float32)
    p4 = p.reshape(jdim, tc, bdim, 2 * cout)
    s1 = jnp.zeros((1, cout), jnp.float32)
    s2 = jnp.zeros((1, cout), jnp.float32)
    for tl in range(tc):
        pt = p4[:, tl].reshape(rows, 2 * cout)
        a = pt[:, :cout]
        c = pt[:, cout:]
        g2d = g_ref[tl].reshape(rows, g_ref.shape[3])
        m_t = jnp.dot(g2d, e_ref[...],
                      preferred_element_type=jnp.float32) * bm_ref[...]
        z = jnp.dot(m_t, a, preferred_element_type=jnp.float32) + c
        z_ref[:, tl] = z.reshape(jdim, bdim, cout).astype(z_ref.dtype)
        s1 = s1 + jnp.sum(z, axis=0, keepdims=True)
        s2 = s2 + jnp.sum(z * z, axis=0, keepdims=True)
    st_ref[...] = jnp.concatenate([s1, s2], axis=0).reshape(1, 2, cout)


def _bn_relu_kernel(n_total, z_ref, st_ref, ga_ref, be_ref, y_ref):
    jdim, tdim, bdim, cout = z_ref.shape
    tot = jnp.sum(st_ref[...], axis=0)
    inv_n = 1.0 / n_total
    mean = tot[0:1] * inv_n
    var = jnp.maximum(tot[1:2] * inv_n - mean * mean, 0.0)
    inv = jax.lax.rsqrt(var + _EPS)
    scale = inv * ga_ref[...]
    shift = be_ref[...] - mean * scale
    z = z_ref[...].reshape(jdim * tdim * bdim, cout)
    y = jnp.maximum(z * scale + shift, 0.0)
    y_ref[...] = y.reshape(jdim, tdim, bdim, cout).astype(y_ref.dtype)


@jax.jit
def _forward(x1, g, w, w1, gamma, beta):
    B, Cin, J, T = x1.shape
    Cout = w.shape[0]

    xp = jnp.transpose(x1, (2, 3, 0, 1))
    gp = jnp.transpose(g, (1, 2, 0, 3))
    wcatt = jnp.concatenate([w.T, w1.T], axis=1)

    col = np.arange(J * B)
    e_sel = jnp.asarray(
        (np.arange(J)[:, None] == col[None, :] // B).astype(np.float32))
    bmask = jnp.asarray(
        (col[:, None] % B == col[None, :] % B).astype(np.float32))

    TC = 5 if T % 5 == 0 else 1
    JC = 5 if J % 5 == 0 else 1
    n_steps = T // TC

    z, stats = pl.pallas_call(
        _proj_mix_kernel,
        out_shape=(
            jax.ShapeDtypeStruct((J, T, B, Cout), jnp.float32),
            jax.ShapeDtypeStruct((n_steps, 2, Cout), jnp.float32),
        ),
        grid=(n_steps,),
        in_specs=[
            pl.BlockSpec((J, TC, B, Cin), lambda t: (0, t, 0, 0)),
            pl.BlockSpec((TC, J, B, J), lambda t: (t, 0, 0, 0)),
            pl.BlockSpec((Cin, 2 * Cout), lambda t: (0, 0)),
            pl.BlockSpec((J, J * B), lambda t: (0, 0)),
            pl.BlockSpec((J * B, J * B), lambda t: (0, 0)),
        ],
        out_specs=(
            pl.BlockSpec((J, TC, B, Cout), lambda t: (0, t, 0, 0)),
            pl.BlockSpec((1, 2, Cout), lambda t: (t, 0, 0)),
        ),
        compiler_params=pltpu.CompilerParams(
            dimension_semantics=("parallel",),
            vmem_limit_bytes=_VMEM_LIMIT,
        ),
    )(xp, gp, wcatt, e_sel, bmask)

    y = pl.pallas_call(
        functools.partial(_bn_relu_kernel, float(B * J * T)),
        out_shape=jax.ShapeDtypeStruct((J, T, B, Cout), x1.dtype),
        grid=(J // JC,),
        in_specs=[
            pl.BlockSpec((JC, T, B, Cout), lambda j: (j, 0, 0, 0)),
            pl.BlockSpec((n_steps, 2, Cout), lambda j: (0, 0, 0)),
            pl.BlockSpec((1, Cout), lambda j: (0, 0)),
            pl.BlockSpec((1, Cout), lambda j: (0, 0)),
        ],
        out_specs=pl.BlockSpec((JC, T, B, Cout), lambda j: (j, 0, 0, 0)),
        compiler_params=pltpu.CompilerParams(
            dimension_semantics=("parallel",),
            vmem_limit_bytes=_VMEM_LIMIT,
        ),
    )(z, stats, gamma.reshape(1, Cout), beta.reshape(1, Cout))

    return jnp.transpose(y, (2, 3, 0, 1))


def kernel(x1, g, w, w1, b1, gamma, beta):
    del b1
    return _forward(x1, g, w, w1, gamma, beta)


# speedup vs baseline: 3.8496x; 1.2966x over previous
import functools

import jax
import jax.numpy as jnp
from jax.experimental import pallas as pl
from jax.experimental.pallas import tpu as pltpu

_EPS = 1e-5
_VMEM_LIMIT = 48 * 1024 * 1024


_TRANS_RHS = (((1,), (1,)), ((), ()))


def _proj_mix_kernel(x_ref, g_ref, w_ref, w1_ref, z_ref, st_ref):
    jdim, tc, bdim, cin = x_ref.shape
    cout = z_ref.shape[3]
    jp = g_ref.shape[3]
    rows = jdim * bdim
    bf = jnp.bfloat16

    erow = jax.lax.broadcasted_iota(jnp.int32, (jp, rows), 0)
    ecol = jax.lax.broadcasted_iota(jnp.int32, (jp, rows), 1)
    e_sel = jnp.where(erow == ecol // bdim, 1.0, 0.0).astype(bf)
    mrow = jax.lax.broadcasted_iota(jnp.int32, (rows, rows), 0)
    mcol = jax.lax.broadcasted_iota(jnp.int32, (rows, rows), 1)
    beq = mrow % bdim == mcol % bdim

    x2d = x_ref[...].reshape(jdim * tc * bdim, cin).astype(bf)
    wb = w_ref[...].astype(bf)
    w1b = w1_ref[...].astype(bf)
    a_all = jax.lax.dot_general(x2d, wb, _TRANS_RHS,
                                preferred_element_type=jnp.float32)
    c_all = jax.lax.dot_general(x2d, w1b, _TRANS_RHS,
                                preferred_element_type=jnp.float32)
    a4 = a_all.reshape(jdim, tc, bdim, cout)
    c4 = c_all.reshape(jdim, tc, bdim, cout)
    s1 = jnp.zeros((1, cout), jnp.float32)
    s2 = jnp.zeros((1, cout), jnp.float32)
    for tl in range(tc):
        a = a4[:, tl].reshape(rows, cout).astype(bf)
        c = c4[:, tl].reshape(rows, cout)
        g2d = g_ref[tl].reshape(rows, jp).astype(bf)
        m_t = jnp.dot(g2d, e_sel, preferred_element_type=jnp.float32)
        m_t = jnp.where(beq, m_t, 0.0).astype(bf)
        z = jnp.dot(m_t, a, preferred_element_type=jnp.float32) + c
        z_ref[:, tl] = z.reshape(jdim, bdim, cout).astype(z_ref.dtype)
        s1 = s1 + jnp.sum(z, axis=0, keepdims=True)
        s2 = s2 + jnp.sum(z * z, axis=0, keepdims=True)
    st_ref[...] = jnp.concatenate([s1, s2], axis=0).reshape(1, 2, cout)


def _bn_relu_kernel(n_total, z_ref, st_ref, ga_ref, be_ref, y_ref):
    jdim, tdim, bdim, cout = z_ref.shape
    tot = jnp.sum(st_ref[...], axis=0)
    inv_n = 1.0 / n_total
    mean = tot[0:1] * inv_n
    var = jnp.maximum(tot[1:2] * inv_n - mean * mean, 0.0)
    inv = jax.lax.rsqrt(var + _EPS)
    scale = inv * ga_ref[...]
    shift = be_ref[...] - mean * scale
    z = z_ref[...].reshape(jdim * tdim * bdim, cout)
    y = jnp.maximum(z * scale + shift, 0.0)
    y_ref[...] = y.reshape(jdim, tdim, bdim, cout).astype(y_ref.dtype)


@jax.jit
def _forward(x1, g, w, w1, gamma, beta):
    B, Cin, J, T = x1.shape
    Cout = w.shape[0]

    xp = jnp.transpose(x1, (2, 3, 0, 1))
    gp = jnp.transpose(g, (1, 2, 0, 3))

    TC = 5 if T % 5 == 0 else 1
    JC = 5 if J % 5 == 0 else 1
    n_steps = T // TC

    z, stats = pl.pallas_call(
        _proj_mix_kernel,
        out_shape=(
            jax.ShapeDtypeStruct((J, T, B, Cout), jnp.float32),
            jax.ShapeDtypeStruct((n_steps, 2, Cout), jnp.float32),
        ),
        grid=(n_steps,),
        in_specs=[
            pl.BlockSpec((J, TC, B, Cin), lambda t: (0, t, 0, 0)),
            pl.BlockSpec((TC, J, B, J), lambda t: (t, 0, 0, 0)),
            pl.BlockSpec((Cout, Cin), lambda t: (0, 0)),
            pl.BlockSpec((Cout, Cin), lambda t: (0, 0)),
        ],
        out_specs=(
            pl.BlockSpec((J, TC, B, Cout), lambda t: (0, t, 0, 0)),
            pl.BlockSpec((1, 2, Cout), lambda t: (t, 0, 0)),
        ),
        compiler_params=pltpu.CompilerParams(
            dimension_semantics=("parallel",),
            vmem_limit_bytes=_VMEM_LIMIT,
        ),
    )(xp, gp, w, w1)

    y = pl.pallas_call(
        functools.partial(_bn_relu_kernel, float(B * J * T)),
        out_shape=jax.ShapeDtypeStruct((J, T, B, Cout), x1.dtype),
        grid=(J // JC,),
        in_specs=[
            pl.BlockSpec((JC, T, B, Cout), lambda j: (j, 0, 0, 0)),
            pl.BlockSpec((n_steps, 2, Cout), lambda j: (0, 0, 0)),
            pl.BlockSpec((1, Cout), lambda j: (0, 0)),
            pl.BlockSpec((1, Cout), lambda j: (0, 0)),
        ],
        out_specs=pl.BlockSpec((JC, T, B, Cout), lambda j: (j, 0, 0, 0)),
        compiler_params=pltpu.CompilerParams(
            dimension_semantics=("parallel",),
            vmem_limit_bytes=_VMEM_LIMIT,
        ),
    )(z, stats, gamma.reshape(1, Cout), beta.reshape(1, Cout))

    return jnp.transpose(y, (2, 3, 0, 1))


def kernel(x1, g, w, w1, b1, gamma, beta):
    del b1
    return _forward(x1, g, w, w1, gamma, beta)


# speedup vs baseline: 3.8622x; 1.0033x over previous
import functools

import jax
import jax.numpy as jnp
from jax.experimental import pallas as pl
from jax.experimental.pallas import tpu as pltpu

_EPS = 1e-5
_VMEM_LIMIT = 48 * 1024 * 1024


_TRANS_RHS = (((1,), (1,)), ((), ()))


def _proj_mix_kernel(x_ref, g_ref, w_ref, w1_ref, z_ref, st_ref):
    jdim, tc, bdim, cin = x_ref.shape
    cout = z_ref.shape[3]
    jp = g_ref.shape[3]
    rows = jdim * bdim
    bf = jnp.bfloat16

    erow = jax.lax.broadcasted_iota(jnp.int32, (jp, rows), 0)
    ecol = jax.lax.broadcasted_iota(jnp.int32, (jp, rows), 1)
    e_sel = jnp.where(erow == ecol // bdim, 1.0, 0.0).astype(bf)
    mrow = jax.lax.broadcasted_iota(jnp.int32, (rows, rows), 0)
    mcol = jax.lax.broadcasted_iota(jnp.int32, (rows, rows), 1)
    beq = mrow % bdim == mcol % bdim

    x2d = x_ref[...].reshape(jdim * tc * bdim, cin).astype(bf)
    wb = w_ref[...].astype(bf)
    w1b = w1_ref[...].astype(bf)
    a_all = jax.lax.dot_general(x2d, wb, _TRANS_RHS,
                                preferred_element_type=jnp.float32)
    c_all = jax.lax.dot_general(x2d, w1b, _TRANS_RHS,
                                preferred_element_type=jnp.float32)
    a4 = a_all.reshape(jdim, tc, bdim, cout)
    c4 = c_all.reshape(jdim, tc, bdim, cout)
    s1 = jnp.zeros((1, cout), jnp.float32)
    s2 = jnp.zeros((1, cout), jnp.float32)
    for tl in range(tc):
        a = a4[:, tl].reshape(rows, cout).astype(bf)
        c = c4[:, tl].reshape(rows, cout)
        g2d = g_ref[tl].reshape(rows, jp).astype(bf)
        m_t = jnp.dot(g2d, e_sel, preferred_element_type=jnp.float32)
        m_t = jnp.where(beq, m_t, 0.0).astype(bf)
        z = jnp.dot(m_t, a, preferred_element_type=jnp.float32) + c
        z_ref[:, tl] = z.reshape(jdim, bdim, cout).astype(z_ref.dtype)
        s1 = s1 + jnp.sum(z, axis=0, keepdims=True)
        s2 = s2 + jnp.sum(z * z, axis=0, keepdims=True)
    st_ref[...] = jnp.concatenate([s1, s2], axis=0).reshape(1, 2, cout)


def _bn_relu_kernel(n_total, z_ref, st_ref, ga_ref, be_ref, y_ref):
    jdim, tdim, bdim, cout = z_ref.shape
    tot = jnp.sum(st_ref[...].astype(jnp.float32), axis=0)
    inv_n = 1.0 / n_total
    mean = tot[0:1] * inv_n
    var = jnp.maximum(tot[1:2] * inv_n - mean * mean, 0.0)
    inv = jax.lax.rsqrt(var + _EPS)
    scale = inv * ga_ref[...]
    shift = be_ref[...] - mean * scale
    z = z_ref[...].reshape(jdim * tdim * bdim, cout).astype(jnp.float32)
    y = jnp.maximum(z * scale + shift, 0.0)
    y_ref[...] = y.reshape(jdim, tdim, bdim, cout).astype(y_ref.dtype)


@jax.jit
def _forward(x1, g, w, w1, gamma, beta):
    B, Cin, J, T = x1.shape
    Cout = w.shape[0]

    xp = jnp.transpose(x1, (2, 3, 0, 1))
    gp = jnp.transpose(g, (1, 2, 0, 3))

    TC = 5 if T % 5 == 0 else 1
    JC = 5 if J % 5 == 0 else 1
    n_steps = T // TC

    z, stats = pl.pallas_call(
        _proj_mix_kernel,
        out_shape=(
            jax.ShapeDtypeStruct((J, T, B, Cout), jnp.bfloat16),
            jax.ShapeDtypeStruct((n_steps, 2, Cout), jnp.float32),
        ),
        grid=(n_steps,),
        in_specs=[
            pl.BlockSpec((J, TC, B, Cin), lambda t: (0, t, 0, 0)),
            pl.BlockSpec((TC, J, B, J), lambda t: (t, 0, 0, 0)),
            pl.BlockSpec((Cout, Cin), lambda t: (0, 0)),
            pl.BlockSpec((Cout, Cin), lambda t: (0, 0)),
        ],
        out_specs=(
            pl.BlockSpec((J, TC, B, Cout), lambda t: (0, t, 0, 0)),
            pl.BlockSpec((1, 2, Cout), lambda t: (t, 0, 0)),
        ),
        compiler_params=pltpu.CompilerParams(
            dimension_semantics=("parallel",),
            vmem_limit_bytes=_VMEM_LIMIT,
        ),
    )(xp, gp, w, w1)

    y = pl.pallas_call(
        functools.partial(_bn_relu_kernel, float(B * J * T)),
        out_shape=jax.ShapeDtypeStruct((J, T, B, Cout), x1.dtype),
        grid=(J // JC,),
        in_specs=[
            pl.BlockSpec((JC, T, B, Cout), lambda j: (j, 0, 0, 0)),
            pl.BlockSpec((n_steps, 2, Cout), lambda j: (0, 0, 0)),
            pl.BlockSpec((1, Cout), lambda j: (0, 0)),
            pl.BlockSpec((1, Cout), lambda j: (0, 0)),
        ],
        out_specs=pl.BlockSpec((JC, T, B, Cout), lambda j: (j, 0, 0, 0)),
        compiler_params=pltpu.CompilerParams(
            dimension_semantics=("parallel",),
            vmem_limit_bytes=_VMEM_LIMIT,
        ),
    )(z, stats, gamma.reshape(1, Cout), beta.reshape(1, Cout))

    return jnp.transpose(y, (2, 3, 0, 1))


def kernel(x1, g, w, w1, b1, gamma, beta):
    del b1
    return _forward(x1, g, w, w1, gamma, beta)
